# baseline (speedup 1.0000x reference)
import functools

import jax
import jax.numpy as jnp
from jax import lax
from jax.experimental import pallas as pl
from jax.experimental.pallas import tpu as pltpu
from jax.experimental.pallas import tpu_sc as plsc

_PF = 4


@functools.lru_cache(maxsize=None)
def _build_sc_kernel(n_edges, n_nodes, d):
    info = plsc.get_sparse_core_info()
    nc, ns = info.num_cores, info.num_subcores
    nw = nc * ns
    assert n_edges % (nw * 16) == 0
    e_per_w = n_edges // nw
    dp = d // 2
    assert d % 2 == 0 and dp % _PF == 0
    nwin = dp // _PF
    n_groups = e_per_w // 16

    mesh = plsc.VectorSubcoreMesh(core_axis_name="c", subcore_axis_name="s")

    @functools.partial(
        pl.kernel,
        mesh=mesh,
        compiler_params=pltpu.CompilerParams(needs_layout_passes=False),
        out_type=jax.ShapeDtypeStruct((n_edges,), jnp.float32),
        scratch_types=[
            pltpu.VMEM((e_per_w,), jnp.int32),
            pltpu.VMEM((e_per_w,), jnp.float32),
            [pltpu.VMEM((n_nodes,), jnp.int32)] * _PF,
            [pltpu.VMEM((n_nodes,), jnp.int32)] * _PF,
            pltpu.SemaphoreType.DMA,
            pltpu.SemaphoreType.DMA,
        ],
    )
    def k(wt_hbm, ip_hbm, out_hbm, ip_v, acc_v, zbs0, zbs1, sem0, sem1):
        wid = lax.axis_index("s") * nc + lax.axis_index("c")
        base = wid * e_per_w
        pltpu.sync_copy(ip_hbm.at[pl.ds(base, e_per_w)], ip_v)

        bufs = (zbs0, zbs1)
        sems = (sem0, sem1)

        def start_window(w):
            b, s = bufs[w % 2], sems[w % 2]
            return [pltpu.async_copy(
                wt_hbm.at[pl.ds((w * _PF + r) * n_nodes, n_nodes)], b[r], s)
                for r in range(_PF)]

        copies = {0: start_window(0)}
        for w in range(nwin):
            for c in copies[w]:
                c.wait()
            if w + 1 < nwin:
                copies[w + 1] = start_window(w + 1)
            zb = bufs[w % 2]
            first = w == 0

            @plsc.parallel_loop(0, n_groups, unroll=10)
            def g_body(g, zb=zb, first=first):
                off = g * 16
                p = ip_v[pl.ds(off, 16)]
                i0g = p & jnp.int32(0xFFFF)
                i1g = lax.shift_right_logical(p, jnp.int32(16))
                if first:
                    a = jnp.zeros((16,), jnp.float32)
                else:
                    a = acc_v[pl.ds(off, 16)]
                for r in range(_PF):
                    wa = plsc.load_gather(zb[r], [i0g])
                    wb = plsc.load_gather(zb[r], [i1g])
                    la = plsc.bitcast(lax.shift_left(wa, jnp.int32(16)),
                                      jnp.float32)
                    lb = plsc.bitcast(lax.shift_left(wb, jnp.int32(16)),
                                      jnp.float32)
                    ha = plsc.bitcast(wa & jnp.int32(-65536), jnp.float32)
                    hb = plsc.bitcast(wb, jnp.float32)
                    a = a + la * lb
                    a = a + ha * hb
                acc_v[pl.ds(off, 16)] = a

        pltpu.sync_copy(acc_v, out_hbm.at[pl.ds(base, e_per_w)])

    return k


def kernel(z, edge_index):
    n_nodes, d = z.shape
    n_edges = edge_index.shape[1]
    idx = edge_index.astype(jnp.int32)
    ip = jnp.bitwise_or(idx[0], jnp.left_shift(idx[1], 16))
    b = lax.bitcast_convert_type(z, jnp.int32)
    rnd = b + jnp.int32(0x7FFF) + (lax.shift_right_logical(b, 16) &
                                   jnp.int32(1))
    lo = lax.shift_right_logical(rnd[:, :d // 2], 16)
    hi = rnd[:, d // 2:] & jnp.int32(-65536)
    w = jnp.bitwise_or(lo, hi)
    wt = jnp.transpose(w).reshape(-1)
    k = _build_sc_kernel(n_edges, n_nodes, d)
    return k(wt, ip)


# speedup vs baseline: 1.0287x; 1.0287x over previous
import functools

import jax
import jax.numpy as jnp
from jax import lax
from jax.experimental import pallas as pl
from jax.experimental.pallas import tpu as pltpu
from jax.experimental.pallas import tpu_sc as plsc

_PF = 4


@functools.lru_cache(maxsize=None)
def _build_sc_kernel(n_edges, n_nodes, d):
    info = plsc.get_sparse_core_info()
    nc, ns = info.num_cores, info.num_subcores
    nw = nc * ns
    assert n_edges % (nw * 16) == 0
    e_per_w = n_edges // nw
    dp = d // 2
    assert d % 2 == 0 and dp % _PF == 0
    nwin = dp // _PF
    n_groups = e_per_w // 16

    mesh = plsc.VectorSubcoreMesh(core_axis_name="c", subcore_axis_name="s")

    @functools.partial(
        pl.kernel,
        mesh=mesh,
        compiler_params=pltpu.CompilerParams(needs_layout_passes=False),
        out_type=jax.ShapeDtypeStruct((n_edges,), jnp.float32),
        scratch_types=[
            pltpu.VMEM((e_per_w,), jnp.int32),
            pltpu.VMEM((e_per_w,), jnp.float32),
            [pltpu.VMEM((n_nodes,), jnp.int32)] * _PF,
            [pltpu.VMEM((n_nodes,), jnp.int32)] * _PF,
            pltpu.SemaphoreType.DMA,
            pltpu.SemaphoreType.DMA,
        ],
    )
    def k(wt_hbm, ip_hbm, out_hbm, ip_v, acc_v, zbs0, zbs1, sem0, sem1):
        wid = lax.axis_index("s") * nc + lax.axis_index("c")
        base = wid * e_per_w
        pltpu.sync_copy(ip_hbm.at[pl.ds(base, e_per_w)], ip_v)

        bufs = (zbs0, zbs1)
        sems = (sem0, sem1)

        def start_window(w):
            b, s = bufs[w % 2], sems[w % 2]
            return [pltpu.async_copy(
                wt_hbm.at[pl.ds((w * _PF + r) * n_nodes, n_nodes)], b[r], s)
                for r in range(_PF)]

        copies = {0: start_window(0)}
        for w in range(nwin):
            for c in copies[w]:
                c.wait()
            if w + 1 < nwin:
                copies[w + 1] = start_window(w + 1)
            zb = bufs[w % 2]
            first = w == 0

            @plsc.parallel_loop(0, n_groups, unroll=5)
            def g_body(g, zb=zb, first=first):
                off = g * 16
                p = ip_v[pl.ds(off, 16)]
                i0g = p & jnp.int32(0xFFFF)
                i1g = lax.shift_right_logical(p, jnp.int32(16))
                if first:
                    a = jnp.zeros((16,), jnp.float32)
                else:
                    a = acc_v[pl.ds(off, 16)]
                for r in range(_PF):
                    wa = plsc.load_gather(zb[r], [i0g])
                    wb = plsc.load_gather(zb[r], [i1g])
                    la = plsc.bitcast(lax.shift_left(wa, jnp.int32(16)),
                                      jnp.float32)
                    lb = plsc.bitcast(lax.shift_left(wb, jnp.int32(16)),
                                      jnp.float32)
                    ha = plsc.bitcast(wa & jnp.int32(-65536), jnp.float32)
                    hb = plsc.bitcast(wb, jnp.float32)
                    a = a + la * lb
                    a = a + ha * hb
                acc_v[pl.ds(off, 16)] = a

        pltpu.sync_copy(acc_v, out_hbm.at[pl.ds(base, e_per_w)])

    return k


def kernel(z, edge_index):
    n_nodes, d = z.shape
    n_edges = edge_index.shape[1]
    idx = edge_index.astype(jnp.int32)
    ip = jnp.bitwise_or(idx[0], jnp.left_shift(idx[1], 16))
    b = lax.bitcast_convert_type(z, jnp.int32)
    rnd = b + jnp.int32(0x7FFF) + (lax.shift_right_logical(b, 16) &
                                   jnp.int32(1))
    lo = lax.shift_right_logical(rnd[:, :d // 2], 16)
    hi = rnd[:, d // 2:] & jnp.int32(-65536)
    w = jnp.bitwise_or(lo, hi)
    wt = jnp.transpose(w).reshape(-1)
    k = _build_sc_kernel(n_edges, n_nodes, d)
    return k(wt, ip)


# speedup vs baseline: 1.0373x; 1.0084x over previous
import functools

import jax
import jax.numpy as jnp
from jax import lax
from jax.experimental import pallas as pl
from jax.experimental.pallas import tpu as pltpu
from jax.experimental.pallas import tpu_sc as plsc

_PF = 5


@functools.lru_cache(maxsize=None)
def _build_sc_kernel(n_edges, n_nodes, d):
    info = plsc.get_sparse_core_info()
    nc, ns = info.num_cores, info.num_subcores
    nw = nc * ns
    assert n_edges % (nw * 16) == 0
    e_per_w = n_edges // nw
    dp = d // 2
    assert d % 2 == 0
    win_rows = [_PF] * (dp // _PF) + ([dp % _PF] if dp % _PF else [])
    win_offs = [sum(win_rows[:i]) for i in range(len(win_rows))]
    nwin = len(win_rows)
    n_groups = e_per_w // 16

    mesh = plsc.VectorSubcoreMesh(core_axis_name="c", subcore_axis_name="s")

    @functools.partial(
        pl.kernel,
        mesh=mesh,
        compiler_params=pltpu.CompilerParams(needs_layout_passes=False),
        out_type=jax.ShapeDtypeStruct((n_edges,), jnp.float32),
        scratch_types=[
            pltpu.VMEM((e_per_w,), jnp.int32),
            pltpu.VMEM((e_per_w,), jnp.float32),
            [pltpu.VMEM((n_nodes,), jnp.int32)] * _PF,
            [pltpu.VMEM((n_nodes,), jnp.int32)] * _PF,
            pltpu.SemaphoreType.DMA,
            pltpu.SemaphoreType.DMA,
        ],
    )
    def k(wt_hbm, ip_hbm, out_hbm, ip_v, acc_v, zbs0, zbs1, sem0, sem1):
        wid = lax.axis_index("s") * nc + lax.axis_index("c")
        base = wid * e_per_w
        pltpu.sync_copy(ip_hbm.at[pl.ds(base, e_per_w)], ip_v)

        bufs = (zbs0, zbs1)
        sems = (sem0, sem1)

        def start_window(w):
            b, s = bufs[w % 2], sems[w % 2]
            return [pltpu.async_copy(
                wt_hbm.at[pl.ds((win_offs[w] + r) * n_nodes, n_nodes)],
                b[r], s) for r in range(win_rows[w])]

        copies = {0: start_window(0)}
        for w in range(nwin):
            for c in copies[w]:
                c.wait()
            if w + 1 < nwin:
                copies[w + 1] = start_window(w + 1)
            zb = bufs[w % 2]
            first = w == 0
            pf_w = win_rows[w]

            @plsc.parallel_loop(0, n_groups, unroll=5)
            def g_body(g, zb=zb, first=first, pf_w=pf_w):
                off = g * 16
                p = ip_v[pl.ds(off, 16)]
                i0g = p & jnp.int32(0xFFFF)
                i1g = lax.shift_right_logical(p, jnp.int32(16))
                if first:
                    a = jnp.zeros((16,), jnp.float32)
                else:
                    a = acc_v[pl.ds(off, 16)]
                for r in range(pf_w):
                    wa = plsc.load_gather(zb[r], [i0g])
                    wb = plsc.load_gather(zb[r], [i1g])
                    la = plsc.bitcast(lax.shift_left(wa, jnp.int32(16)),
                                      jnp.float32)
                    lb = plsc.bitcast(lax.shift_left(wb, jnp.int32(16)),
                                      jnp.float32)
                    ha = plsc.bitcast(wa & jnp.int32(-65536), jnp.float32)
                    hb = plsc.bitcast(wb, jnp.float32)
                    a = a + la * lb
                    a = a + ha * hb
                acc_v[pl.ds(off, 16)] = a

        pltpu.sync_copy(acc_v, out_hbm.at[pl.ds(base, e_per_w)])

    return k


def kernel(z, edge_index):
    n_nodes, d = z.shape
    n_edges = edge_index.shape[1]
    idx = edge_index.astype(jnp.int32)
    ip = jnp.bitwise_or(idx[0], jnp.left_shift(idx[1], 16))
    b = lax.bitcast_convert_type(z, jnp.int32)
    rnd = b + jnp.int32(0x7FFF) + (lax.shift_right_logical(b, 16) &
                                   jnp.int32(1))
    lo = lax.shift_right_logical(rnd[:, :d // 2], 16)
    hi = rnd[:, d // 2:] & jnp.int32(-65536)
    w = jnp.bitwise_or(lo, hi)
    wt = jnp.transpose(w).reshape(-1)
    k = _build_sc_kernel(n_edges, n_nodes, d)
    return k(wt, ip)


# speedup vs baseline: 1.0419x; 1.0044x over previous
import functools

import jax
import jax.numpy as jnp
from jax import lax
from jax.experimental import pallas as pl
from jax.experimental.pallas import tpu as pltpu
from jax.experimental.pallas import tpu_sc as plsc

_PF = 5


@functools.lru_cache(maxsize=None)
def _build_sc_kernel(n_edges, n_nodes, d):
    info = plsc.get_sparse_core_info()
    nc, ns = info.num_cores, info.num_subcores
    nw = nc * ns
    assert n_edges % (nw * 16) == 0
    e_per_w = n_edges // nw
    dp = d // 2
    assert d % 2 == 0
    win_rows = [_PF] * (dp // _PF) + ([dp % _PF] if dp % _PF else [])
    win_offs = [sum(win_rows[:i]) for i in range(len(win_rows))]
    nwin = len(win_rows)
    n_groups = e_per_w // 16

    mesh = plsc.VectorSubcoreMesh(core_axis_name="c", subcore_axis_name="s")

    @functools.partial(
        pl.kernel,
        mesh=mesh,
        compiler_params=pltpu.CompilerParams(needs_layout_passes=False),
        out_type=jax.ShapeDtypeStruct((n_edges,), jnp.float32),
        scratch_types=[
            pltpu.VMEM((e_per_w,), jnp.int32),
            pltpu.VMEM((e_per_w,), jnp.float32),
            [pltpu.VMEM((n_nodes,), jnp.int32)] * _PF,
            [pltpu.VMEM((n_nodes,), jnp.int32)] * _PF,
            pltpu.SemaphoreType.DMA,
            pltpu.SemaphoreType.DMA,
            pltpu.SemaphoreType.DMA,
        ],
    )
    def k(wt_hbm, ip_hbm, out_hbm, ip_v, acc_v, zbs0, zbs1, sem0, sem1,
          sem_ip):
        wid = lax.axis_index("s") * nc + lax.axis_index("c")
        base = wid * e_per_w

        bufs = (zbs0, zbs1)
        sems = (sem0, sem1)

        def start_window(w):
            b, s = bufs[w % 2], sems[w % 2]
            return [pltpu.async_copy(
                wt_hbm.at[pl.ds((win_offs[w] + r) * n_nodes, n_nodes)],
                b[r], s) for r in range(win_rows[w])]

        copies = {0: start_window(0)}
        cp_ip = pltpu.async_copy(ip_hbm.at[pl.ds(base, e_per_w)], ip_v,
                                 sem_ip)
        cp_ip.wait()
        for w in range(nwin):
            for c in copies[w]:
                c.wait()
            if w + 1 < nwin:
                copies[w + 1] = start_window(w + 1)
            zb = bufs[w % 2]
            first = w == 0
            pf_w = win_rows[w]

            @plsc.parallel_loop(0, n_groups, unroll=5)
            def g_body(g, zb=zb, first=first, pf_w=pf_w):
                off = g * 16
                p = ip_v[pl.ds(off, 16)]
                i0g = p & jnp.int32(0xFFFF)
                i1g = lax.shift_right_logical(p, jnp.int32(16))
                if first:
                    a = jnp.zeros((16,), jnp.float32)
                else:
                    a = acc_v[pl.ds(off, 16)]
                for r in range(pf_w):
                    wa = plsc.load_gather(zb[r], [i0g])
                    wb = plsc.load_gather(zb[r], [i1g])
                    la = plsc.bitcast(lax.shift_left(wa, jnp.int32(16)),
                                      jnp.float32)
                    lb = plsc.bitcast(lax.shift_left(wb, jnp.int32(16)),
                                      jnp.float32)
                    ha = plsc.bitcast(wa & jnp.int32(-65536), jnp.float32)
                    hb = plsc.bitcast(wb, jnp.float32)
                    a = a + la * lb
                    a = a + ha * hb
                acc_v[pl.ds(off, 16)] = a

        pltpu.sync_copy(acc_v, out_hbm.at[pl.ds(base, e_per_w)])

    return k


def kernel(z, edge_index):
    n_nodes, d = z.shape
    n_edges = edge_index.shape[1]
    idx = edge_index.astype(jnp.int32)
    ip = jnp.bitwise_or(idx[0], jnp.left_shift(idx[1], 16))
    b = lax.bitcast_convert_type(z, jnp.int32)
    rnd = b + jnp.int32(0x7FFF) + (lax.shift_right_logical(b, 16) &
                                   jnp.int32(1))
    lo = lax.shift_right_logical(rnd[:, :d // 2], 16)
    hi = rnd[:, d // 2:] & jnp.int32(-65536)
    w = jnp.bitwise_or(lo, hi)
    wt = jnp.transpose(w).reshape(-1)
    k = _build_sc_kernel(n_edges, n_nodes, d)
    return k(wt, ip)
